# baseline (speedup 1.0000x reference)
import functools

import jax
import jax.numpy as jnp
from jax import lax
from jax.experimental import pallas as pl
from jax.experimental.pallas import tpu as pltpu
from jax.experimental.pallas import tpu_sc as plsc

N_U = 10000
N_E = 10000
E = 320000
D = 128

NS = 16
EDGES_PER_TILE = E // NS
CHUNK = 80
NCHUNK = EDGES_PER_TILE // CHUNK
G = 16
NG = CHUNK // G
NSETS = 3
NITER = (NCHUNK + NSETS - 1) // NSETS
OBLK = 40
NBLK = N_E // OBLK
BLK_ITERS = (NBLK + NS - 1) // NS

_mesh = plsc.VectorSubcoreMesh(core_axis_name="c", subcore_axis_name="s")


def _set_types():
    return [
        pltpu.VMEM((CHUNK, D), jnp.float32),
        pltpu.VMEM((CHUNK,), jnp.int32),
        pltpu.VMEM((CHUNK,), jnp.int32),
        pltpu.VMEM((CHUNK,), jnp.int32),
        pltpu.VMEM((CHUNK,), jnp.float32),
        pltpu.SemaphoreType.DMA,
        pltpu.SemaphoreType.DMA,
        pltpu.SemaphoreType.DMA,
    ]


@functools.partial(
    pl.kernel,
    out_type=(
        jax.ShapeDtypeStruct((N_E, D), jnp.float32),
        jax.ShapeDtypeStruct((N_U, D), jnp.float32),
    ),
    mesh=_mesh,
    scratch_types=[
        pltpu.VMEM((OBLK, D), jnp.float32),
        pltpu.VMEM((OBLK, D), jnp.float32),
        pltpu.SemaphoreType.DMA,
        pltpu.SemaphoreType.DMA,
        pltpu.VMEM_SHARED((N_E, D), jnp.float32),
    ] + _set_types() + _set_types() + _set_types(),
)
def _agg(user_emb, entity_emb, rows_hbm, cols_hbm, vals_hbm,
         out_entity, out_user, obuf, obuf2, osem, osem2, acc,
         rb0, gi0, si0, sb0, va0, isem0, gsem0, ssem0,
         rb1, gi1, si1, sb1, va1, isem1, gsem1, ssem1,
         rb2, gi2, si2, sb2, va2, isem2, gsem2, ssem2):
    cid = lax.axis_index("c")
    sid = lax.axis_index("s")
    sets = (
        (rb0, gi0, si0, sb0, va0, isem0, gsem0, ssem0),
        (rb1, gi1, si1, sb1, va1, isem1, gsem1, ssem1),
        (rb2, gi2, si2, sb2, va2, isem2, gsem2, ssem2),
    )

    def run(table, gidx_hbm, sidx_hbm, out_hbm):
        base0 = sid * EDGES_PER_TILE

        zeros16 = jnp.zeros((16,), jnp.float32)

        def zb(i, carry):
            obuf[i // 8, pl.ds((i % 8) * 16, 16)] = zeros16
            return carry

        lax.fori_loop(0, OBLK * (D // 16), zb, 0)

        def zcopy(j, carry):
            blk = sid + j * NS

            @pl.when(blk < NBLK)
            def _():
                pltpu.async_copy(
                    obuf, acc.at[pl.ds(blk * OBLK, OBLK)], osem)

            return carry

        lax.fori_loop(0, BLK_ITERS, zcopy, 0)

        def zdrain(j, carry):
            blk = sid + j * NS

            @pl.when(blk < NBLK)
            def _():
                pltpu.make_async_copy(
                    obuf, acc.at[pl.ds(blk * OBLK, OBLK)], osem).wait()

            return carry

        lax.fori_loop(0, BLK_ITERS, zdrain, 0)
        plsc.subcore_barrier()

        def idx_start(c, s):
            rbuf, gi, si, sb, va, isem, gsem, ssem = s
            base = base0 + c * CHUNK
            pltpu.async_copy(gidx_hbm.at[pl.ds(base, CHUNK)], gi, isem)
            pltpu.async_copy(sidx_hbm.at[pl.ds(base, CHUNK)], si, isem)
            pltpu.async_copy(vals_hbm.at[pl.ds(base, CHUNK)], va, isem)

        def idx_wait(s):
            rbuf, gi, si, sb, va, isem, gsem, ssem = s
            pltpu.make_async_copy(
                gidx_hbm.at[pl.ds(0, CHUNK)], gi, isem).wait()
            pltpu.make_async_copy(
                sidx_hbm.at[pl.ds(0, CHUNK)], si, isem).wait()
            pltpu.make_async_copy(
                vals_hbm.at[pl.ds(0, CHUNK)], va, isem).wait()

        def gfire(s):
            rbuf, gi, si, sb, va, isem, gsem, ssem = s
            pltpu.async_copy(table.at[gi], rbuf, gsem)

        def gwait(s):
            rbuf, gi, si, sb, va, isem, gsem, ssem = s
            pltpu.make_async_copy(table.at[gi], rbuf, gsem).wait()

        def scale_fire(c, s):
            del c
            rbuf, gi, si, sb, va, isem, gsem, ssem = s
            for g in range(NG):
                sl = pl.ds(g * G, G)
                sb[sl] = si[sl]

            for g in range(NG):
                vals16 = va[pl.ds(g * G, G)]
                for i in range(G):
                    vv = jnp.full((16,), vals16[i])
                    row = g * G + i
                    for d in range(D // 16):
                        sl = pl.ds(d * 16, 16)
                        rbuf[row, sl] = rbuf[row, sl] * vv
            pltpu.async_copy(rbuf, acc.at[sb], ssem, add=True)

        def sdrain(s):
            rbuf, gi, si, sb, va, isem, gsem, ssem = s
            pltpu.make_async_copy(rbuf, acc.at[sb], ssem).wait()

        idx_start(0, sets[0])
        idx_start(1, sets[1])
        idx_start(2, sets[2])
        idx_wait(sets[0])
        gfire(sets[0])

        def half(c, k):
            x = sets[k]
            y = sets[(k + 1) % NSETS]

            @pl.when(c < NCHUNK)
            def _():
                @pl.when(c >= 2)
                def _():
                    sdrain(y)

                @pl.when(c + 1 < NCHUNK)
                def _():
                    idx_wait(y)
                    gfire(y)

                gwait(x)
                scale_fire(c, x)

                @pl.when(c + 3 < NCHUNK)
                def _():
                    idx_start(c + 3, x)

        def pipe(j, carry):
            c = 3 * j
            half(c, 0)
            half(c + 1, 1)
            half(c + 2, 2)
            return carry

        lax.fori_loop(0, NITER, pipe, 0)

        sdrain(sets[(NCHUNK - 2) % NSETS])
        sdrain(sets[(NCHUNK - 1) % NSETS])
        plsc.subcore_barrier()

        def oread(blk, buf, sem):
            pltpu.async_copy(acc.at[pl.ds(blk * OBLK, OBLK)], buf, sem)

        def oread_wait(buf, sem):
            pltpu.make_async_copy(acc.at[pl.ds(0, OBLK)], buf, sem).wait()

        def owrite(blk, buf, sem):
            pltpu.async_copy(buf, out_hbm.at[pl.ds(blk * OBLK, OBLK)], sem)

        def owrite_wait(buf, sem):
            pltpu.make_async_copy(
                buf, out_hbm.at[pl.ds(0, OBLK)], sem).wait()

        def ohalf(blk, buf, sem, first):
            @pl.when(blk < NBLK)
            def _():
                @pl.when(jnp.logical_not(first))
                def _():
                    owrite_wait(buf, sem)

                oread(blk, buf, sem)
                oread_wait(buf, sem)
                owrite(blk, buf, sem)

        def ocopy(j, carry):
            blk = sid + 2 * j * NS
            ohalf(blk, obuf, osem, j == 0)
            ohalf(blk + NS, obuf2, osem2, j == 0)
            return carry

        lax.fori_loop(0, (BLK_ITERS + 1) // 2, ocopy, 0)

        owrite_wait(obuf, osem)
        owrite_wait(obuf2, osem2)

    @pl.when(cid == 0)
    def _():
        run(user_emb, rows_hbm, cols_hbm, out_entity)

    @pl.when(cid == 1)
    def _():
        run(entity_emb, cols_hbm, rows_hbm, out_user)


def kernel(user_emb, entity_emb, interact_rows, interact_cols, interact_vals):
    return _agg(user_emb, entity_emb, interact_rows, interact_cols,
                interact_vals)


# speedup vs baseline: 1.3823x; 1.3823x over previous
import functools

import jax
import jax.numpy as jnp
from jax import lax
from jax.experimental import pallas as pl
from jax.experimental.pallas import tpu as pltpu
from jax.experimental.pallas import tpu_sc as plsc

N_U = 10000
N_E = 10000
E = 320000
D = 128

NS = 16
EDGES_PER_TILE = E // NS
CHUNK = 80
NCHUNK = EDGES_PER_TILE // CHUNK
G = 16
NG = CHUNK // G
NSETS = 3
NITER = (NCHUNK + NSETS - 1) // NSETS
OBLK = 40
NBLK = N_E // OBLK
BLK_ITERS = (NBLK + NS - 1) // NS

_mesh = plsc.VectorSubcoreMesh(core_axis_name="c", subcore_axis_name="s")


def _set_types():
    return [
        pltpu.VMEM((CHUNK, D), jnp.float32),
        pltpu.VMEM((CHUNK,), jnp.int32),
        pltpu.VMEM((CHUNK,), jnp.int32),
        pltpu.VMEM((CHUNK,), jnp.int32),
        pltpu.VMEM((CHUNK,), jnp.float32),
        pltpu.SemaphoreType.DMA,
        pltpu.SemaphoreType.DMA,
        pltpu.SemaphoreType.DMA,
    ]


@functools.partial(
    pl.kernel,
    out_type=(
        jax.ShapeDtypeStruct((N_E, D), jnp.float32),
        jax.ShapeDtypeStruct((N_U, D), jnp.float32),
    ),
    mesh=_mesh,
    scratch_types=[
        pltpu.VMEM((OBLK, D), jnp.float32),
        pltpu.VMEM((OBLK, D), jnp.float32),
        pltpu.VMEM((3 * CHUNK,), jnp.int32),
        pltpu.SemaphoreType.DMA,
        pltpu.SemaphoreType.DMA,
        pltpu.VMEM_SHARED((N_E, D), jnp.float32),
    ] + _set_types() + _set_types() + _set_types(),
)
def _agg(user_emb, entity_emb, rows_hbm, cols_hbm, vals_hbm,
         out_entity, out_user, obuf, obuf2, dum, osem, osem2, acc,
         rb0, gi0, si0, sb0, va0, isem0, gsem0, ssem0,
         rb1, gi1, si1, sb1, va1, isem1, gsem1, ssem1,
         rb2, gi2, si2, sb2, va2, isem2, gsem2, ssem2):
    cid = lax.axis_index("c")
    sid = lax.axis_index("s")
    sets = (
        (rb0, gi0, si0, sb0, va0, isem0, gsem0, ssem0),
        (rb1, gi1, si1, sb1, va1, isem1, gsem1, ssem1),
        (rb2, gi2, si2, sb2, va2, isem2, gsem2, ssem2),
    )

    def run(table, gidx_hbm, sidx_hbm, out_hbm):
        base0 = sid * EDGES_PER_TILE

        def prologue():
            idx_start(0, sets[0])
            idx_start(1, sets[1])
            idx_start(2, sets[2])
            idx_wait(sets[0])
            gfire(sets[0])

        def idx_start(c, s):
            rbuf, gi, si, sb, va, isem, gsem, ssem = s
            base = base0 + c * CHUNK
            pltpu.async_copy(gidx_hbm.at[pl.ds(base, CHUNK)], gi, isem)
            pltpu.async_copy(sidx_hbm.at[pl.ds(base, CHUNK)], si, isem)
            pltpu.async_copy(vals_hbm.at[pl.ds(base, CHUNK)], va, isem)

        def idx_wait(s):
            rbuf, gi, si, sb, va, isem, gsem, ssem = s
            pltpu.make_async_copy(
                gidx_hbm.at[pl.ds(0, 3 * CHUNK)], dum, isem).wait()

        def gfire(s):
            rbuf, gi, si, sb, va, isem, gsem, ssem = s
            pltpu.async_copy(table.at[gi], rbuf, gsem)

        def gwait(s):
            rbuf, gi, si, sb, va, isem, gsem, ssem = s
            pltpu.make_async_copy(table.at[gi], rbuf, gsem).wait()

        def scale_fire(c, s):
            del c
            rbuf, gi, si, sb, va, isem, gsem, ssem = s
            for g in range(NG):
                sl = pl.ds(g * G, G)
                sb[sl] = si[sl]

            def group(g, carry):
                vals16 = va[pl.ds(g * G, G)]
                for i in range(G):
                    vv = jnp.full((16,), vals16[i])
                    row = g * G + i
                    for d in range(D // 16):
                        sl = pl.ds(d * 16, 16)
                        rbuf[row, sl] = rbuf[row, sl] * vv
                return carry

            lax.fori_loop(0, NG, group, 0)
            pltpu.async_copy(rbuf, acc.at[sb], ssem, add=True)

        def sdrain(s):
            rbuf, gi, si, sb, va, isem, gsem, ssem = s
            pltpu.make_async_copy(rbuf, acc.at[sb], ssem).wait()


        zeros16 = jnp.zeros((16,), jnp.float32)

        def zb(i, carry):
            obuf[i // 8, pl.ds((i % 8) * 16, 16)] = zeros16
            return carry

        lax.fori_loop(0, OBLK * (D // 16), zb, 0)
        prologue()

        def zcopy(j, carry):
            blk = sid + j * NS

            @pl.when(blk < NBLK)
            def _():
                pltpu.async_copy(
                    obuf, acc.at[pl.ds(blk * OBLK, OBLK)], osem)

            return carry

        lax.fori_loop(0, BLK_ITERS, zcopy, 0)

        def zdrain(j, carry):
            blk = sid + j * NS

            @pl.when(blk < NBLK)
            def _():
                pltpu.make_async_copy(
                    obuf, acc.at[pl.ds(blk * OBLK, OBLK)], osem).wait()

            return carry

        lax.fori_loop(0, BLK_ITERS, zdrain, 0)
        plsc.subcore_barrier()

        def half(c, k):
            x = sets[k]
            y = sets[(k + 1) % NSETS]

            @pl.when(c < NCHUNK)
            def _():
                @pl.when(c >= 2)
                def _():
                    sdrain(y)

                @pl.when(c + 1 < NCHUNK)
                def _():
                    idx_wait(y)
                    gfire(y)

                gwait(x)
                scale_fire(c, x)

                @pl.when(c + 3 < NCHUNK)
                def _():
                    idx_start(c + 3, x)

        def pipe(j, carry):
            c = 3 * j
            half(c, 0)
            half(c + 1, 1)
            half(c + 2, 2)
            return carry

        lax.fori_loop(0, NITER, pipe, 0)

        sdrain(sets[(NCHUNK - 2) % NSETS])
        sdrain(sets[(NCHUNK - 1) % NSETS])
        plsc.subcore_barrier()

        def oread(blk, buf, sem):
            pltpu.async_copy(acc.at[pl.ds(blk * OBLK, OBLK)], buf, sem)

        def oread_wait(buf, sem):
            pltpu.make_async_copy(acc.at[pl.ds(0, OBLK)], buf, sem).wait()

        def owrite(blk, buf, sem):
            pltpu.async_copy(buf, out_hbm.at[pl.ds(blk * OBLK, OBLK)], sem)

        def owrite_wait(buf, sem):
            pltpu.make_async_copy(
                buf, out_hbm.at[pl.ds(0, OBLK)], sem).wait()

        def ohalf(blk, buf, sem, first):
            @pl.when(blk < NBLK)
            def _():
                @pl.when(jnp.logical_not(first))
                def _():
                    owrite_wait(buf, sem)

                oread(blk, buf, sem)
                oread_wait(buf, sem)
                owrite(blk, buf, sem)

        def ocopy(j, carry):
            blk = sid + 2 * j * NS
            ohalf(blk, obuf, osem, j == 0)
            ohalf(blk + NS, obuf2, osem2, j == 0)
            return carry

        lax.fori_loop(0, (BLK_ITERS + 1) // 2, ocopy, 0)

        owrite_wait(obuf, osem)
        owrite_wait(obuf2, osem2)

    @pl.when(cid == 0)
    def _():
        run(user_emb, rows_hbm, cols_hbm, out_entity)

    @pl.when(cid == 1)
    def _():
        run(entity_emb, cols_hbm, rows_hbm, out_user)


def kernel(user_emb, entity_emb, interact_rows, interact_cols, interact_vals):
    return _agg(user_emb, entity_emb, interact_rows, interact_cols,
                interact_vals)
